# Initial kernel scaffold; baseline (speedup 1.0000x reference)
import jax
import jax.numpy as jnp
from jax.experimental import pallas as pl


def kernel(sequence_tensor, positions):
    raise NotImplementedError("write your pallas kernel here")


# speedup vs baseline: 1.0266x; 1.0266x over previous
import functools

import jax
import jax.numpy as jnp
from jax import lax
from jax.experimental import pallas as pl
from jax.experimental.pallas import tpu as pltpu
from jax.experimental.pallas import tpu_sc as plsc


def _make_gather(n_rows_pad, d, n_workers, n_chunks, chunk):
    mesh = plsc.VectorSubcoreMesh(core_axis_name="c", subcore_axis_name="s")
    nc = 2

    @functools.partial(
        pl.kernel,
        mesh=mesh,
        out_type=jax.ShapeDtypeStruct((n_rows_pad, d), jnp.float32),
        scratch_types=[
            pltpu.VMEM((n_chunks, chunk), jnp.int32),
            pltpu.VMEM((chunk, d), jnp.float32),
            pltpu.SemaphoreType.DMA,
        ],
    )
    def gather_k(table_hbm, idx_hbm, out_hbm, idx_v, rows_v, sem):
        wid = lax.axis_index("s") * nc + lax.axis_index("c")
        base = wid * (n_chunks * chunk)
        pltpu.sync_copy(idx_hbm.at[wid], idx_v)
        for c in range(n_chunks):
            pltpu.async_copy(table_hbm.at[idx_v.at[c]], rows_v, sem).wait()
            pltpu.sync_copy(rows_v, out_hbm.at[pl.ds(base + c * chunk, chunk)])

    return gather_k


def kernel(sequence_tensor, positions):
    bt, seq, d = sequence_tensor.shape
    _, p = positions.shape
    n_rows = bt * p

    n_workers = 32
    chunk = 32
    per_w = -(-n_rows // (n_workers * chunk)) * chunk
    n_chunks = per_w // chunk
    n_rows_pad = per_w * n_workers

    table = sequence_tensor.reshape(bt * seq, d)
    idx = (positions.astype(jnp.int32) + jnp.arange(bt, dtype=jnp.int32)[:, None] * seq)
    idx = idx.reshape(-1)
    idx = jnp.pad(idx, (0, n_rows_pad - n_rows))
    idx = idx.reshape(n_workers, n_chunks, chunk)

    out = _make_gather(n_rows_pad, d, n_workers, n_chunks, chunk)(table, idx)
    return out[:n_rows].reshape(bt, p, d)
